# baseline (speedup 1.0000x reference)
import jax
import jax.numpy as jnp
from jax.experimental import pallas as pl
from jax.experimental.pallas import tpu as pltpu

_DIM = 768
_NH = 12
_HD = 64
_E = 8
_HID = 256
_SH = 768
_EPS = 1e-05
_TL = 256
_TQ = 512
_L = 2048
_BF = jnp.bfloat16
_LOG2E = 1.4426950408889634
_HP = 2 * _HD


def _rms(x, w):
    return x * jax.lax.rsqrt(jnp.mean(x * x, axis=-1, keepdims=True) + _EPS) * w


def _dot_t(a, b):
    return jax.lax.dot_general(a, b, (((1,), (1,)), ((), ())),
                               preferred_element_type=jnp.float32)


def _dot(a, b):
    return jax.lax.dot_general(a, b, (((1,), (0,)), ((), ())),
                               preferred_element_type=jnp.float32)


def _qkv_body(x_ref, n1_ref, wq_ref, wk_ref, wv_ref, q_ref, k_ref, v_ref,
              wq_s, wk_s, wv_s):
    @pl.when(pl.program_id(0) == 0)
    def _cast():
        wq_s[...] = wq_ref[...].astype(_BF)
        wk_s[...] = wk_ref[...].astype(_BF)
        wv_s[...] = wv_ref[...].astype(_BF)

    xn = _rms(x_ref[...], n1_ref[...]).astype(_BF)
    q_ref[...] = _dot_t(xn, wq_s[...]).astype(_BF)
    k_ref[...] = _dot_t(xn, wk_s[...]).astype(_BF)
    v_ref[...] = _dot_t(xn, wv_s[...]).astype(_BF)


def _rope_tables(nrows, row0):
    pos = row0 + jax.lax.broadcasted_iota(
        jnp.int32, (nrows, _HD // 2), 0).astype(jnp.float32)
    j = jax.lax.broadcasted_iota(
        jnp.int32, (nrows, _HD // 2), 1).astype(jnp.float32)
    inv = jax.lax.exp2(j * ((-2.0 / _HD) * (jnp.log(10000.0) * _LOG2E)))
    fr = pos * inv
    cos = jnp.cos(fr).astype(_BF)
    sin = jnp.sin(fr).astype(_BF)
    cos4 = jnp.concatenate([cos, cos, cos, cos], axis=1)
    sin4 = jnp.concatenate([sin, sin, sin, sin], axis=1)
    return cos4, sin4


def _rot_pair(t):
    d = _HD // 2
    return jnp.concatenate(
        [-t[:, d:2 * d], t[:, :d], -t[:, 3 * d:], t[:, 2 * d:3 * d]], axis=1)


def _attn_body(q_ref, k_ref, v_ref, o_ref, kr_s):
    iq = pl.program_id(1)

    @pl.when(iq == 0)
    def _rope_k():
        cos, sin = _rope_tables(_L, 0.0)
        k2 = k_ref[...]
        kr_s[...] = k2 * cos + _rot_pair(k2) * sin

    cos, sin = _rope_tables(_TQ, (iq * _TQ).astype(jnp.float32))
    q2 = q_ref[...]
    q2 = (q2 * cos + _rot_pair(q2) * sin) * _BF(_HD ** -0.5 * _LOG2E)
    k2 = kr_s[...]
    v2 = v_ref[...]
    outs = []
    for jh in range(2):
        sl = slice(jh * _HD, (jh + 1) * _HD)
        s = _dot_t(q2[:, sl], k2[:, sl]).astype(_BF)
        m = jnp.max(s, axis=-1, keepdims=True)
        p = jax.lax.exp2(s - m)
        l = jnp.sum(p, axis=-1, keepdims=True).astype(jnp.float32)
        o = _dot(p, v2[:, sl])
        outs.append((o * (1.0 / l)).astype(_BF))
    o_ref[...] = jnp.concatenate(outs, axis=-1)


def _block2_body(a_ref, x_ref, wo_ref, n2_ref, gw_ref, s1_ref, s2_ref,
                 f1_ref, s3_ref, f2_ref, o_ref,
                 wo_s, s1_s, s2_s, f1_s, s3_s, f2_s):
    @pl.when(pl.program_id(0) == 0)
    def _cast():
        wo_s[...] = wo_ref[...].astype(_BF)
        s1_s[...] = s1_ref[...].astype(_BF)
        s2_s[...] = s2_ref[...].astype(_BF)
        f1_s[...] = f1_ref[...].astype(_BF)
        s3_s[...] = s3_ref[...].astype(_BF)
        f2_s[...] = f2_ref[...].astype(_BF)

    h = x_ref[...] + _dot_t(a_ref[...], wo_s[...])
    hn = _rms(h, n2_ref[...])
    hnb = hn.astype(_BF)
    logits = _dot_t(hnb, gw_ref[...].astype(_BF))
    idx = jax.lax.broadcasted_iota(jnp.int32, logits.shape, 1)
    m1 = jnp.max(logits, axis=-1, keepdims=True)
    a1 = jnp.min(jnp.where(logits == m1, idx, _E), axis=-1, keepdims=True)
    oh1 = idx == a1
    masked = jnp.where(oh1, -jnp.inf, logits)
    m2 = jnp.max(masked, axis=-1, keepdims=True)
    a2 = jnp.min(jnp.where(masked == m2, idx, _E), axis=-1, keepdims=True)
    oh2 = idx == a2
    w1 = jax.lax.logistic(m1 - m2)
    wd = (jnp.where(oh1, w1, 0.0) + jnp.where(oh2, 1.0 - w1, 0.0)).astype(_BF)
    lane_e = jax.lax.broadcasted_iota(jnp.int32, (_E, _E * _HID), 1) // _HID
    row_e = jax.lax.broadcasted_iota(jnp.int32, (_E, _E * _HID), 0)
    rmat = (lane_e == row_e).astype(_BF)
    wexp = _dot(wd, rmat).astype(_BF)
    g = (jax.nn.silu(_dot_t(hnb, s1_s[...])) *
         _dot_t(hnb, s2_s[...])).astype(_BF)
    acc = h + _dot_t(g, s3_s[...])
    he = (jax.nn.silu(_dot_t(hnb, f1_s[...])) * wexp).astype(_BF)
    for e in range(_E):
        acc = acc + _dot_t(he[:, e * _HID:(e + 1) * _HID], f2_s[e])
    o_ref[...] = acc


def kernel(x, wq, wk, wv, wo, norm1_w, norm2_w, gate_w, fc1_w, fc2_w,
           sh1_w, sh2_w, sh3_w):
    B, L, D = x.shape
    xf = x.reshape(L, D)
    NQ = L // _TL
    n1 = norm1_w.reshape(1, D)
    n2 = norm2_w.reshape(1, D)
    fc1c = fc1_w.reshape(_E * _HID, D)

    q, k, v = pl.pallas_call(
        _qkv_body,
        grid=(NQ,),
        in_specs=[
            pl.BlockSpec((_TL, D), lambda i: (i, 0)),
            pl.BlockSpec((1, D), lambda i: (0, 0)),
            pl.BlockSpec((D, D), lambda i: (0, 0)),
            pl.BlockSpec((D, D), lambda i: (0, 0)),
            pl.BlockSpec((D, D), lambda i: (0, 0)),
        ],
        out_specs=[pl.BlockSpec((_TL, D), lambda i: (i, 0))] * 3,
        out_shape=[jax.ShapeDtypeStruct((L, D), _BF)] * 3,
        scratch_shapes=[pltpu.VMEM((D, D), _BF)] * 3,
    )(xf, n1, wq, wk, wv)

    a = pl.pallas_call(
        _attn_body,
        grid=(_NH // 2, L // _TQ),
        in_specs=[
            pl.BlockSpec((_TQ, _HP), lambda h, i: (i, h)),
            pl.BlockSpec((L, _HP), lambda h, i: (0, h)),
            pl.BlockSpec((L, _HP), lambda h, i: (0, h)),
        ],
        out_specs=pl.BlockSpec((_TQ, _HP), lambda h, i: (i, h)),
        out_shape=jax.ShapeDtypeStruct((L, D), _BF),
        scratch_shapes=[pltpu.VMEM((_L, _HP), _BF)],
    )(q, k, v)

    out = pl.pallas_call(
        _block2_body,
        grid=(NQ,),
        in_specs=[
            pl.BlockSpec((_TL, D), lambda i: (i, 0)),
            pl.BlockSpec((_TL, D), lambda i: (i, 0)),
            pl.BlockSpec((D, D), lambda i: (0, 0)),
            pl.BlockSpec((1, D), lambda i: (0, 0)),
            pl.BlockSpec((_E, D), lambda i: (0, 0)),
            pl.BlockSpec((_SH, D), lambda i: (0, 0)),
            pl.BlockSpec((_SH, D), lambda i: (0, 0)),
            pl.BlockSpec((_E * _HID, D), lambda i: (0, 0)),
            pl.BlockSpec((D, _SH), lambda i: (0, 0)),
            pl.BlockSpec((_E, D, _HID), lambda i: (0, 0, 0)),
        ],
        out_specs=pl.BlockSpec((_TL, D), lambda i: (i, 0)),
        out_shape=jax.ShapeDtypeStruct((L, D), jnp.float32),
        scratch_shapes=[
            pltpu.VMEM((D, D), _BF),
            pltpu.VMEM((_SH, D), _BF),
            pltpu.VMEM((_SH, D), _BF),
            pltpu.VMEM((_E * _HID, D), _BF),
            pltpu.VMEM((D, _SH), _BF),
            pltpu.VMEM((_E, D, _HID), _BF),
        ],
    )(a, xf, wo, n2, gate_w, sh1_w, sh2_w, fc1c, sh3_w, fc2_w)

    return out.reshape(B, L, D)


# speedup vs baseline: 1.2244x; 1.2244x over previous
import jax
import jax.numpy as jnp
from jax.experimental import pallas as pl
from jax.experimental.pallas import tpu as pltpu

_DIM = 768
_NH = 12
_HD = 64
_E = 8
_HID = 256
_SH = 768
_EPS = 1e-05
_TL = 256
_TQ = 512
_L = 2048
_BF = jnp.bfloat16
_LOG2E = 1.4426950408889634
_HP = 2 * _HD


def _rms(x, w):
    return x * jax.lax.rsqrt(jnp.mean(x * x, axis=-1, keepdims=True) + _EPS) * w


def _dot_t(a, b):
    return jax.lax.dot_general(a, b, (((1,), (1,)), ((), ())),
                               preferred_element_type=jnp.float32)


def _dot(a, b):
    return jax.lax.dot_general(a, b, (((1,), (0,)), ((), ())),
                               preferred_element_type=jnp.float32)


def _qkv_body(x_ref, n1_ref, wq_ref, wk_ref, wv_ref, cos_ref, sin_ref,
              q_ref, k_ref, v_ref, wq_s, wk_s, wv_s, wqr_s, wkr_s):
    @pl.when(pl.program_id(0) == 0)
    def _cast():
        wq_s[...] = wq_ref[...].astype(_BF)
        wk_s[...] = wk_ref[...].astype(_BF)
        wv_s[...] = wv_ref[...].astype(_BF)
        d = _HD // 2
        for hh in range(_NH):
            r0 = hh * _HD
            wqr_s[r0:r0 + d, :] = -wq_ref[r0 + d:r0 + _HD, :].astype(_BF)
            wqr_s[r0 + d:r0 + _HD, :] = wq_ref[r0:r0 + d, :].astype(_BF)
            wkr_s[r0:r0 + d, :] = -wk_ref[r0 + d:r0 + _HD, :].astype(_BF)
            wkr_s[r0 + d:r0 + _HD, :] = wk_ref[r0:r0 + d, :].astype(_BF)

    xn = _rms(x_ref[...], n1_ref[...]).astype(_BF)
    cos = cos_ref[...]
    sin = sin_ref[...]
    q = _dot_t(xn, wq_s[...]).astype(_BF)
    qr = _dot_t(xn, wqr_s[...]).astype(_BF)
    q_ref[...] = q * cos + qr * sin
    k = _dot_t(xn, wk_s[...]).astype(_BF)
    kr = _dot_t(xn, wkr_s[...]).astype(_BF)
    k_ref[...] = k * cos + kr * sin
    v_ref[...] = _dot_t(xn, wv_s[...]).astype(_BF)


def _attn_body(q_ref, k_ref, v_ref, o_ref):
    q2 = q_ref[...] * _BF(_HD ** -0.5 * _LOG2E)
    k2 = k_ref[...]
    v2 = v_ref[...]
    outs = []
    for jh in range(2):
        sl = slice(jh * _HD, (jh + 1) * _HD)
        s = _dot_t(q2[:, sl], k2[:, sl]).astype(_BF)
        m = jnp.max(s, axis=-1, keepdims=True)
        p = jax.lax.exp2(s - m)
        l = jnp.sum(p, axis=-1, keepdims=True).astype(jnp.float32)
        o = _dot(p, v2[:, sl])
        outs.append((o * (1.0 / l)).astype(_BF))
    o_ref[...] = jnp.concatenate(outs, axis=-1)


def _block2_body(a_ref, x_ref, wo_ref, n2_ref, gw_ref, s1_ref, s2_ref,
                 f1_ref, s3_ref, f2_ref, o_ref,
                 wo_s, s1_s, s2_s, f1_s, s3_s, f2_s):
    @pl.when(pl.program_id(0) == 0)
    def _cast():
        wo_s[...] = wo_ref[...].astype(_BF)
        s1_s[...] = s1_ref[...].astype(_BF)
        s2_s[...] = s2_ref[...].astype(_BF)
        f1_s[...] = f1_ref[...].astype(_BF)
        s3_s[...] = s3_ref[...].astype(_BF)
        f2_s[...] = f2_ref[...].astype(_BF)

    h = x_ref[...] + _dot_t(a_ref[...], wo_s[...])
    hn = _rms(h, n2_ref[...])
    hnb = hn.astype(_BF)
    logits = _dot_t(hnb, gw_ref[...].astype(_BF))
    idx = jax.lax.broadcasted_iota(jnp.int32, logits.shape, 1)
    m1 = jnp.max(logits, axis=-1, keepdims=True)
    a1 = jnp.min(jnp.where(logits == m1, idx, _E), axis=-1, keepdims=True)
    oh1 = idx == a1
    masked = jnp.where(oh1, -jnp.inf, logits)
    m2 = jnp.max(masked, axis=-1, keepdims=True)
    a2 = jnp.min(jnp.where(masked == m2, idx, _E), axis=-1, keepdims=True)
    oh2 = idx == a2
    w1 = jax.lax.logistic(m1 - m2)
    wd = (jnp.where(oh1, w1, 0.0) + jnp.where(oh2, 1.0 - w1, 0.0)).astype(_BF)
    lane_e = jax.lax.broadcasted_iota(jnp.int32, (_E, _E * _HID), 1) // _HID
    row_e = jax.lax.broadcasted_iota(jnp.int32, (_E, _E * _HID), 0)
    rmat = (lane_e == row_e).astype(_BF)
    wexp = _dot(wd, rmat).astype(_BF)
    g = (jax.nn.silu(_dot_t(hnb, s1_s[...])) *
         _dot_t(hnb, s2_s[...])).astype(_BF)
    acc = h + _dot_t(g, s3_s[...])
    he = (jax.nn.silu(_dot_t(hnb, f1_s[...])) * wexp).astype(_BF)
    for e in range(_E):
        acc = acc + _dot_t(he[:, e * _HID:(e + 1) * _HID], f2_s[e])
    o_ref[...] = acc


def kernel(x, wq, wk, wv, wo, norm1_w, norm2_w, gate_w, fc1_w, fc2_w,
           sh1_w, sh2_w, sh3_w):
    B, L, D = x.shape
    xf = x.reshape(L, D)
    NQ = L // _TL
    n1 = norm1_w.reshape(1, D)
    n2 = norm2_w.reshape(1, D)
    fc1c = fc1_w.reshape(_E * _HID, D)

    inv = 1.0 / (10000.0 ** (jnp.arange(0, _HD, 2, dtype=jnp.float32) / _HD))
    t = jnp.arange(L, dtype=jnp.float32)
    freqs = jnp.outer(t, inv)
    emb = jnp.concatenate([freqs, freqs], axis=-1)
    cos = jnp.tile(jnp.cos(emb), (1, _NH)).astype(_BF)
    sin = jnp.tile(jnp.sin(emb), (1, _NH)).astype(_BF)

    q, k, v = pl.pallas_call(
        _qkv_body,
        grid=(NQ,),
        in_specs=[
            pl.BlockSpec((_TL, D), lambda i: (i, 0)),
            pl.BlockSpec((1, D), lambda i: (0, 0)),
            pl.BlockSpec((D, D), lambda i: (0, 0)),
            pl.BlockSpec((D, D), lambda i: (0, 0)),
            pl.BlockSpec((D, D), lambda i: (0, 0)),
            pl.BlockSpec((_TL, D), lambda i: (i, 0)),
            pl.BlockSpec((_TL, D), lambda i: (i, 0)),
        ],
        out_specs=[pl.BlockSpec((_TL, D), lambda i: (i, 0))] * 3,
        out_shape=[jax.ShapeDtypeStruct((L, D), _BF)] * 3,
        scratch_shapes=[pltpu.VMEM((D, D), _BF)] * 5,
    )(xf, n1, wq, wk, wv, cos, sin)

    a = pl.pallas_call(
        _attn_body,
        grid=(_NH // 2, L // _TQ),
        in_specs=[
            pl.BlockSpec((_TQ, _HP), lambda h, i: (i, h)),
            pl.BlockSpec((L, _HP), lambda h, i: (0, h)),
            pl.BlockSpec((L, _HP), lambda h, i: (0, h)),
        ],
        out_specs=pl.BlockSpec((_TQ, _HP), lambda h, i: (i, h)),
        out_shape=jax.ShapeDtypeStruct((L, D), _BF),
    )(q, k, v)

    out = pl.pallas_call(
        _block2_body,
        grid=(NQ,),
        in_specs=[
            pl.BlockSpec((_TL, D), lambda i: (i, 0)),
            pl.BlockSpec((_TL, D), lambda i: (i, 0)),
            pl.BlockSpec((D, D), lambda i: (0, 0)),
            pl.BlockSpec((1, D), lambda i: (0, 0)),
            pl.BlockSpec((_E, D), lambda i: (0, 0)),
            pl.BlockSpec((_SH, D), lambda i: (0, 0)),
            pl.BlockSpec((_SH, D), lambda i: (0, 0)),
            pl.BlockSpec((_E * _HID, D), lambda i: (0, 0)),
            pl.BlockSpec((D, _SH), lambda i: (0, 0)),
            pl.BlockSpec((_E, D, _HID), lambda i: (0, 0, 0)),
        ],
        out_specs=pl.BlockSpec((_TL, D), lambda i: (i, 0)),
        out_shape=jax.ShapeDtypeStruct((L, D), jnp.float32),
        scratch_shapes=[
            pltpu.VMEM((D, D), _BF),
            pltpu.VMEM((_SH, D), _BF),
            pltpu.VMEM((_SH, D), _BF),
            pltpu.VMEM((_E * _HID, D), _BF),
            pltpu.VMEM((D, _SH), _BF),
            pltpu.VMEM((_E, D, _HID), _BF),
        ],
    )(a, xf, wo, n2, gate_w, sh1_w, sh2_w, fc1c, sh3_w, fc2_w)

    return out.reshape(B, L, D)


# speedup vs baseline: 1.2482x; 1.0194x over previous
import jax
import jax.numpy as jnp
from jax.experimental import pallas as pl
from jax.experimental.pallas import tpu as pltpu

_DIM = 768
_NH = 12
_HD = 64
_E = 8
_HID = 256
_SH = 768
_EPS = 1e-05
_TL = 256
_TQ = 1024
_L = 2048
_BF = jnp.bfloat16
_LOG2E = 1.4426950408889634
_HP = 2 * _HD


def _rms(x, w):
    return x * jax.lax.rsqrt(jnp.mean(x * x, axis=-1, keepdims=True) + _EPS) * w


def _dot_t(a, b):
    return jax.lax.dot_general(a, b, (((1,), (1,)), ((), ())),
                               preferred_element_type=jnp.float32)


def _dot(a, b):
    return jax.lax.dot_general(a, b, (((1,), (0,)), ((), ())),
                               preferred_element_type=jnp.float32)


def _qkv_body(x_ref, n1_ref, wq_ref, wk_ref, wv_ref, cos_ref, sin_ref,
              q_ref, k_ref, v_ref, wq_s, wk_s, wv_s, wqr_s, wkr_s):
    @pl.when(pl.program_id(0) == 0)
    def _cast():
        wq_s[...] = wq_ref[...].astype(_BF)
        wk_s[...] = wk_ref[...].astype(_BF)
        wv_s[...] = wv_ref[...].astype(_BF)
        d = _HD // 2
        for hh in range(_NH):
            r0 = hh * _HD
            wqr_s[r0:r0 + d, :] = -wq_ref[r0 + d:r0 + _HD, :].astype(_BF)
            wqr_s[r0 + d:r0 + _HD, :] = wq_ref[r0:r0 + d, :].astype(_BF)
            wkr_s[r0:r0 + d, :] = -wk_ref[r0 + d:r0 + _HD, :].astype(_BF)
            wkr_s[r0 + d:r0 + _HD, :] = wk_ref[r0:r0 + d, :].astype(_BF)

    xn = _rms(x_ref[...], n1_ref[...]).astype(_BF)
    cos = cos_ref[...]
    sin = sin_ref[...]
    q = _dot_t(xn, wq_s[...]).astype(_BF)
    qr = _dot_t(xn, wqr_s[...]).astype(_BF)
    q_ref[...] = q * cos + qr * sin
    k = _dot_t(xn, wk_s[...]).astype(_BF)
    kr = _dot_t(xn, wkr_s[...]).astype(_BF)
    k_ref[...] = k * cos + kr * sin
    v_ref[...] = _dot_t(xn, wv_s[...]).astype(_BF)


def _attn_body(q_ref, k_ref, v_ref, o_ref):
    q2 = q_ref[...] * _BF(_HD ** -0.5 * _LOG2E)
    k2 = k_ref[...]
    v2 = v_ref[...]
    outs = []
    for jh in range(2):
        sl = slice(jh * _HD, (jh + 1) * _HD)
        s = _dot_t(q2[:, sl], k2[:, sl]).astype(_BF)
        m = jnp.max(s, axis=-1, keepdims=True)
        p = jax.lax.exp2(s - m)
        l = jnp.sum(p, axis=-1, keepdims=True).astype(jnp.float32)
        o = _dot(p, v2[:, sl])
        outs.append((o * (1.0 / l)).astype(_BF))
    o_ref[...] = jnp.concatenate(outs, axis=-1)


def _block2_body(a_ref, x_ref, wo_ref, n2_ref, gw_ref, s1_ref, s2_ref,
                 f1_ref, s3_ref, f2_ref, o_ref,
                 wo_s, s1_s, s2_s, f1_s, s3_s, f2_s):
    @pl.when(pl.program_id(0) == 0)
    def _cast():
        wo_s[...] = wo_ref[...].astype(_BF)
        s1_s[...] = s1_ref[...].astype(_BF)
        s2_s[...] = s2_ref[...].astype(_BF)
        f1_s[...] = f1_ref[...].astype(_BF)
        s3_s[...] = s3_ref[...].astype(_BF)
        f2_s[...] = f2_ref[...].astype(_BF)

    h = x_ref[...] + _dot_t(a_ref[...], wo_s[...])
    hn = _rms(h, n2_ref[...])
    hnb = hn.astype(_BF)
    logits = _dot_t(hnb, gw_ref[...].astype(_BF))
    idx = jax.lax.broadcasted_iota(jnp.int32, logits.shape, 1)
    m1 = jnp.max(logits, axis=-1, keepdims=True)
    a1 = jnp.min(jnp.where(logits == m1, idx, _E), axis=-1, keepdims=True)
    oh1 = idx == a1
    masked = jnp.where(oh1, -jnp.inf, logits)
    m2 = jnp.max(masked, axis=-1, keepdims=True)
    a2 = jnp.min(jnp.where(masked == m2, idx, _E), axis=-1, keepdims=True)
    oh2 = idx == a2
    w1 = jax.lax.logistic(m1 - m2)
    wd = (jnp.where(oh1, w1, 0.0) + jnp.where(oh2, 1.0 - w1, 0.0)).astype(_BF)
    lane_e = jax.lax.broadcasted_iota(jnp.int32, (_E, _E * _HID), 1) // _HID
    row_e = jax.lax.broadcasted_iota(jnp.int32, (_E, _E * _HID), 0)
    rmat = (lane_e == row_e).astype(_BF)
    wexp = _dot(wd, rmat).astype(_BF)
    g = (jax.nn.silu(_dot_t(hnb, s1_s[...])) *
         _dot_t(hnb, s2_s[...])).astype(_BF)
    acc = h + _dot_t(g, s3_s[...])
    he = (jax.nn.silu(_dot_t(hnb, f1_s[...])) * wexp).astype(_BF)
    for e in range(_E):
        acc = acc + _dot_t(he[:, e * _HID:(e + 1) * _HID], f2_s[e])
    o_ref[...] = acc


def kernel(x, wq, wk, wv, wo, norm1_w, norm2_w, gate_w, fc1_w, fc2_w,
           sh1_w, sh2_w, sh3_w):
    B, L, D = x.shape
    xf = x.reshape(L, D)
    NQ = L // _TL
    n1 = norm1_w.reshape(1, D)
    n2 = norm2_w.reshape(1, D)
    fc1c = fc1_w.reshape(_E * _HID, D)

    inv = 1.0 / (10000.0 ** (jnp.arange(0, _HD, 2, dtype=jnp.float32) / _HD))
    t = jnp.arange(L, dtype=jnp.float32)
    freqs = jnp.outer(t, inv)
    emb = jnp.concatenate([freqs, freqs], axis=-1)
    cos = jnp.tile(jnp.cos(emb), (1, _NH)).astype(_BF)
    sin = jnp.tile(jnp.sin(emb), (1, _NH)).astype(_BF)

    q, k, v = pl.pallas_call(
        _qkv_body,
        grid=(NQ,),
        in_specs=[
            pl.BlockSpec((_TL, D), lambda i: (i, 0)),
            pl.BlockSpec((1, D), lambda i: (0, 0)),
            pl.BlockSpec((D, D), lambda i: (0, 0)),
            pl.BlockSpec((D, D), lambda i: (0, 0)),
            pl.BlockSpec((D, D), lambda i: (0, 0)),
            pl.BlockSpec((_TL, D), lambda i: (i, 0)),
            pl.BlockSpec((_TL, D), lambda i: (i, 0)),
        ],
        out_specs=[pl.BlockSpec((_TL, D), lambda i: (i, 0))] * 3,
        out_shape=[jax.ShapeDtypeStruct((L, D), _BF)] * 3,
        scratch_shapes=[pltpu.VMEM((D, D), _BF)] * 5,
    )(xf, n1, wq, wk, wv, cos, sin)

    a = pl.pallas_call(
        _attn_body,
        grid=(_NH // 2, L // _TQ),
        in_specs=[
            pl.BlockSpec((_TQ, _HP), lambda h, i: (i, h)),
            pl.BlockSpec((L, _HP), lambda h, i: (0, h)),
            pl.BlockSpec((L, _HP), lambda h, i: (0, h)),
        ],
        out_specs=pl.BlockSpec((_TQ, _HP), lambda h, i: (i, h)),
        out_shape=jax.ShapeDtypeStruct((L, D), _BF),
    )(q, k, v)

    out = pl.pallas_call(
        _block2_body,
        grid=(NQ,),
        in_specs=[
            pl.BlockSpec((_TL, D), lambda i: (i, 0)),
            pl.BlockSpec((_TL, D), lambda i: (i, 0)),
            pl.BlockSpec((D, D), lambda i: (0, 0)),
            pl.BlockSpec((1, D), lambda i: (0, 0)),
            pl.BlockSpec((_E, D), lambda i: (0, 0)),
            pl.BlockSpec((_SH, D), lambda i: (0, 0)),
            pl.BlockSpec((_SH, D), lambda i: (0, 0)),
            pl.BlockSpec((_E * _HID, D), lambda i: (0, 0)),
            pl.BlockSpec((D, _SH), lambda i: (0, 0)),
            pl.BlockSpec((_E, D, _HID), lambda i: (0, 0, 0)),
        ],
        out_specs=pl.BlockSpec((_TL, D), lambda i: (i, 0)),
        out_shape=jax.ShapeDtypeStruct((L, D), jnp.float32),
        scratch_shapes=[
            pltpu.VMEM((D, D), _BF),
            pltpu.VMEM((_SH, D), _BF),
            pltpu.VMEM((_SH, D), _BF),
            pltpu.VMEM((_E * _HID, D), _BF),
            pltpu.VMEM((D, _SH), _BF),
            pltpu.VMEM((_E, D, _HID), _BF),
        ],
    )(a, xf, wo, n2, gate_w, sh1_w, sh2_w, fc1c, sh3_w, fc2_w)

    return out.reshape(B, L, D)
